# baseline (speedup 1.0000x reference)
import functools

import jax
import jax.numpy as jnp
from jax import lax
from jax.experimental import pallas as pl
from jax.experimental.pallas import tpu as pltpu
from jax.experimental.pallas import tpu_sc as plsc

EMBED_DIM = 32
NUM_CORES = 2
NUM_SUBCORES = 16
NUM_WORKERS = NUM_CORES * NUM_SUBCORES
CHUNK = 832
NUM_SLOTS = 4


def _make_gather(total_b):
    assert total_b % (NUM_WORKERS * CHUNK) == 0
    b_per_w = total_b // NUM_WORKERS
    n_chunks = b_per_w // CHUNK
    assert n_chunks >= NUM_SLOTS
    mesh = plsc.VectorSubcoreMesh(
        core_axis_name="c", subcore_axis_name="s",
        num_cores=NUM_CORES, num_subcores=NUM_SUBCORES)

    @functools.partial(
        pl.kernel,
        mesh=mesh,
        compiler_params=pltpu.CompilerParams(use_tc_tiling_on_sc=False),
        out_type=jax.ShapeDtypeStruct((total_b, EMBED_DIM), jnp.float32),
        scratch_types=[
            pltpu.VMEM((b_per_w,), jnp.int32),
            pltpu.VMEM((NUM_SLOTS, CHUNK, EMBED_DIM), jnp.float32),
            pltpu.SemaphoreType.DMA,
            pltpu.SemaphoreType.DMA,
            pltpu.SemaphoreType.DMA,
            pltpu.SemaphoreType.DMA,
            pltpu.SemaphoreType.DMA,
        ],
    )
    def gather_kernel(ids_hbm, table_hbm, out_hbm, idx_v, rows_v,
                      sem_g0, sem_g1, sem_g2, sem_g3, sem_o):
        wid = lax.axis_index("s") * NUM_CORES + lax.axis_index("c")
        base = wid * b_per_w
        sems = (sem_g0, sem_g1, sem_g2, sem_g3)

        pltpu.sync_copy(ids_hbm.at[wid], idx_v)

        def gather_copy(i):
            s = i % NUM_SLOTS
            return pltpu.make_async_copy(
                table_hbm.at[idx_v.at[pl.ds(i * CHUNK, CHUNK)]],
                rows_v.at[s], sems[s])

        def out_copy(i):
            s = i % NUM_SLOTS
            return pltpu.make_async_copy(
                rows_v.at[s],
                out_hbm.at[pl.ds(base + i * CHUNK, CHUNK)],
                sem_o)

        for i in range(NUM_SLOTS - 1):
            gather_copy(i).start()
        for i in range(n_chunks):
            gather_copy(i).wait()
            out_copy(i).start()
            j = i + NUM_SLOTS - 1
            if j < n_chunks:
                if i >= 1:
                    out_copy(i - 1).wait()
                gather_copy(j).start()
        for k in range(max(0, n_chunks - NUM_SLOTS), n_chunks):
            out_copy(k).wait()

    return gather_kernel


def kernel(inputs, table):
    flat_ids = inputs.reshape(-1).astype(jnp.int32)
    total_b = flat_ids.shape[0]
    ids2 = flat_ids.reshape(NUM_WORKERS, total_b // NUM_WORKERS)
    flat_out = _make_gather(total_b)(ids2, table)
    return flat_out.reshape(inputs.shape + (EMBED_DIM,))
